# Initial kernel scaffold; baseline (speedup 1.0000x reference)
import jax
import jax.numpy as jnp
from jax.experimental import pallas as pl


def kernel(inr_features, anchor_idx, pos_idx, neg_idx):
    raise NotImplementedError("write your pallas kernel here")


# speedup vs baseline: 1.7778x; 1.7778x over previous
import functools

import jax
import jax.numpy as jnp
from jax import lax
from jax.experimental import pallas as pl
from jax.experimental.pallas import tpu as pltpu
from jax.experimental.pallas import tpu_sc as plsc

NC = 2
NS = 16
NW = NC * NS
L = 16


def _sc_partial_sumsq(f2d, a_idx, p_idx, n_idx, N, C):
    BN, D = f2d.shape
    B = BN // N
    T = a_idx.shape[0]
    TW = T // NW
    NCHUNK = TW // C
    NL = 3 * B

    mesh = plsc.VectorSubcoreMesh(
        core_axis_name="c", subcore_axis_name="s",
        num_cores=NC, num_subcores=NS)

    @functools.partial(
        pl.kernel,
        out_type=[jax.ShapeDtypeStruct((T, L), jnp.float32),
                  jax.ShapeDtypeStruct((T, L), jnp.float32)],
        mesh=mesh,
        scratch_types=(
            [pltpu.VMEM((TW,), jnp.int32) for _ in range(3)]
            + [pltpu.VMEM((TW,), jnp.int32) for _ in range(NL)]
            + [pltpu.VMEM((C, D), jnp.float32) for _ in range(NL)]
            + [pltpu.VMEM((TW, L), jnp.float32),
               pltpu.VMEM((TW, L), jnp.float32),
               pltpu.SemaphoreType.DMA]
        ),
        compiler_params=pltpu.CompilerParams(use_tc_tiling_on_sc=False),
    )
    def k(f_hbm, ai_hbm, pi_hbm, ni_hbm, oap_hbm, oan_hbm, *sc):
        idx_v = sc[0:3]
        rix_v = sc[3:3 + NL]
        rows_v = sc[3 + NL:3 + 2 * NL]
        oap_v, oan_v, sem = sc[3 + 2 * NL:]
        wid = lax.axis_index("s") * NC + lax.axis_index("c")
        base = wid * TW

        cps = [pltpu.async_copy(h.at[pl.ds(base, TW)], idx_v[i], sem)
               for i, h in enumerate((ai_hbm, pi_hbm, ni_hbm))]
        for cp in cps:
            cp.wait()

        def build(g, _):
            for l in range(3):
                v = idx_v[l][pl.ds(g * L, L)]
                for b in range(B):
                    rix_v[l * B + b][pl.ds(g * L, L)] = v + b * N
            return 0

        lax.fori_loop(0, TW // L, build, 0)

        for ci in range(NCHUNK):
            cps = [pltpu.async_copy(
                       f_hbm.at[rix_v[lb].at[pl.ds(ci * C, C)]],
                       rows_v[lb], sem)
                   for lb in range(NL)]
            for cp in cps:
                cp.wait()

            def trip(t, _):
                acc_ap = jnp.zeros((L,), jnp.float32)
                acc_an = jnp.zeros((L,), jnp.float32)
                for b in range(B):
                    for cc in range(D // L):
                        sl = pl.ds(cc * L, L)
                        va = rows_v[b][t, sl]
                        vp = rows_v[B + b][t, sl]
                        vn = rows_v[2 * B + b][t, sl]
                        dap = va - vp
                        dan = va - vn
                        acc_ap = acc_ap + dap * dap
                        acc_an = acc_an + dan * dan
                oap_v[ci * C + t, :] = acc_ap
                oan_v[ci * C + t, :] = acc_an
                return 0

            lax.fori_loop(0, C, trip, 0)

        pltpu.sync_copy(oap_v, oap_hbm.at[pl.ds(base, TW)])
        pltpu.sync_copy(oan_v, oan_hbm.at[pl.ds(base, TW)])

    return k(f2d, a_idx, p_idx, n_idx)


def _tc_finish(ap2, an2):
    def body(ap_ref, an_ref, o_ref):
        d_ap = jnp.sqrt(jnp.sum(ap_ref[...], axis=1))
        d_an = jnp.sqrt(jnp.sum(an_ref[...], axis=1))
        o_ref[0, 0] = jnp.sum(jnp.maximum(d_ap - d_an, 0.0))

    out = pl.pallas_call(
        body,
        out_shape=jax.ShapeDtypeStruct((1, 1), jnp.float32),
        out_specs=pl.BlockSpec(memory_space=pltpu.SMEM),
    )(ap2, an2)
    return out[0, 0]


def kernel(inr_features, anchor_idx, pos_idx, neg_idx):
    B, N, D = inr_features.shape
    f2d = inr_features.reshape(B * N, D)
    ai = anchor_idx.astype(jnp.int32)
    pi = pos_idx.astype(jnp.int32)
    ni = neg_idx.astype(jnp.int32)
    ap2, an2 = _sc_partial_sumsq(f2d, ai, pi, ni, N, C=64)
    return _tc_finish(ap2, an2)
